# baseline (speedup 1.0000x reference)
import math
import functools

import jax
import jax.numpy as jnp
from jax.experimental import pallas as pl

_K = 6
_M = 2
_L_TRAIN = 2048
_L_MAX = 1000000
_BETA = 0.5
_RAMP_LAMBDA = 0.2
_TAU = 64.0
_C_SCALE = 0.01
_WIDTH_MIN = 32.0
_WIDTH_MAX = 256.0
_DELTA_STAR_MAX = float(_L_TRAIN - 1)
_W_MIN = 2.0 * math.pi / _L_MAX
_W_MAX = 2.0 * math.pi / _L_TRAIN

_TI = 256


def _softplus(x):
    return jnp.maximum(x, 0.0) + jnp.log1p(jnp.exp(-jnp.abs(x)))


def _bias_kernel(q_ref, w1t_ref, b1_ref, w2t_ref, b2_ref, om_ref, out_ref):
    ti = q_ref.shape[2]
    L = out_ref.shape[3]
    i0 = pl.program_id(1) * ti

    qb = q_ref[0, 0]
    h1 = jnp.dot(qb, w1t_ref[...], preferred_element_type=jnp.float32)
    h1 = h1 + b1_ref[...]
    h1 = h1 * jax.nn.sigmoid(h1)
    p = jnp.dot(h1, w2t_ref[...], preferred_element_type=jnp.float32)
    p = p + b2_ref[...]

    om = om_ref[...]
    d_logit = p[:, 0:_M]
    w_logit = p[:, _M:2 * _M]
    c_raw = p[:, 2 * _M:3 * _M]
    pi_logit = p[:, 3 * _M:4 * _M]
    slope = p[:, 4 * _M:4 * _M + 1]

    delta_star = jax.nn.sigmoid(d_logit) * _DELTA_STAR_MAX
    width = _WIDTH_MIN + jax.nn.sigmoid(w_logit) * (_WIDTH_MAX - _WIDTH_MIN)
    c = _C_SCALE * _softplus(c_raw)
    pi_max = jnp.max(pi_logit, axis=1, keepdims=True)
    pi_e = jnp.exp(pi_logit - pi_max)
    pi = pi_e / jnp.sum(pi_e, axis=1, keepdims=True)
    s = _softplus(slope) / _TAU

    ii_col = (i0 + jax.lax.broadcasted_iota(jnp.int32, (ti, 1), 0)
              ).astype(jnp.float32)
    th_row = ii_col * om

    a_cos = jnp.zeros((ti, _K), dtype=jnp.float32)
    a_sin = jnp.zeros((ti, _K), dtype=jnp.float32)
    b0 = jnp.zeros((ti, 1), dtype=jnp.float32)
    for m in range(_M):
        g_m = jnp.exp(-0.5 * _BETA * (om * width[:, m:m + 1]) ** 2)
        alpha_m = om * delta_star[:, m:m + 1]
        amp_m = (pi[:, m:m + 1] * c[:, m:m + 1]) * g_m
        a_cos = a_cos + amp_m * jnp.cos(th_row - alpha_m)
        a_sin = a_sin + amp_m * jnp.sin(th_row - alpha_m)
        b0 = b0 + jnp.sum(amp_m * jnp.cos(alpha_m), axis=1, keepdims=True)

    a_mat = jnp.concatenate([a_cos, a_sin], axis=1)

    jj_col = jax.lax.broadcasted_iota(jnp.int32, (L, 1), 0).astype(jnp.float32)
    th_all = jj_col * om
    c_tab = jnp.concatenate([jnp.cos(th_all), jnp.sin(th_all)], axis=1)

    bias = jax.lax.dot_general(
        a_mat, c_tab, (((1,), (1,)), ((), ())),
        preferred_element_type=jnp.float32)

    ii_i = i0 + jax.lax.broadcasted_iota(jnp.int32, (ti, L), 0)
    jj_i = jax.lax.broadcasted_iota(jnp.int32, (ti, L), 1)
    delta = (ii_i - jj_i).astype(jnp.float32)
    bias = bias - b0 - _RAMP_LAMBDA * s * jnp.maximum(delta, 0.0)
    out_ref[0, 0] = jnp.where(delta >= 0.0, bias, 0.0)


@jax.jit
def kernel(q, W1, b1, W2, b2):
    B, H, L, D = q.shape
    out_dim = W2.shape[0]
    hidden = W1.shape[0]

    omegas = jnp.logspace(math.log10(_W_MIN), math.log10(_W_MAX), _K,
                          dtype=jnp.float32).reshape(1, _K)
    pad = 16 - out_dim
    w2t = jnp.pad(W2, ((0, pad), (0, 0))).T
    b2p = jnp.pad(b2, (0, pad)).reshape(1, 16)
    w1t = W1.T
    b1r = b1.reshape(1, hidden)

    n_i = L // _TI
    grid = (H, n_i)

    out = pl.pallas_call(
        _bias_kernel,
        grid=grid,
        in_specs=[
            pl.BlockSpec((1, 1, _TI, D), lambda h, i: (0, h, i, 0)),
            pl.BlockSpec((D, hidden), lambda h, i: (0, 0)),
            pl.BlockSpec((1, hidden), lambda h, i: (0, 0)),
            pl.BlockSpec((hidden, 16), lambda h, i: (0, 0)),
            pl.BlockSpec((1, 16), lambda h, i: (0, 0)),
            pl.BlockSpec((1, _K), lambda h, i: (0, 0)),
        ],
        out_specs=pl.BlockSpec((1, 1, _TI, L), lambda h, i: (0, h, i, 0)),
        out_shape=jax.ShapeDtypeStruct((B, H, L, L), jnp.float32),
    )(q, w1t, b1r, w2t, b2p, omegas)
    return out
